# baseline (speedup 1.0000x reference)
import functools

import jax
import jax.numpy as jnp
from jax.experimental import pallas as pl
from jax.experimental.pallas import tpu as pltpu

D = 2048
NE = 8
K = 2
ED = 8192
KE = 256
NSTEPS = ED // KE

_NT = (((1,), (1,)), ((), ()))


def _moe_kernel(x_ref, wg_ref, wup_ref, wdn_ref, out_ref, scale_ref):
    ke = pl.program_id(0)

    @pl.when(ke == 0)
    def _gate():
        logits = jax.lax.dot_general(
            x_ref[...], wg_ref[...], _NT, preferred_element_type=jnp.float32
        )
        m = jnp.max(logits, axis=1, keepdims=True)
        e = jnp.exp(logits - m)
        se = jnp.sum(e, axis=1, keepdims=True)
        m1 = jnp.max(e, axis=1, keepdims=True)
        a1 = jnp.argmax(e, axis=1, keepdims=True)
        lane = jax.lax.broadcasted_iota(jnp.int32, e.shape, 1)
        m2 = jnp.max(jnp.where(lane == a1, -jnp.inf, e), axis=1, keepdims=True)
        s = (m1 + m2) / se
        scale_ref[...] = s / (s + 1e-9)
        out_ref[...] = jnp.zeros_like(out_ref)

    h = jax.lax.dot_general(
        x_ref[...],
        wup_ref[...].astype(jnp.bfloat16),
        _NT,
        preferred_element_type=jnp.float32,
    )
    h = (h * jax.lax.logistic(h)).astype(jnp.bfloat16)
    contrib = jax.lax.dot_general(
        h,
        wdn_ref[...].astype(jnp.bfloat16),
        _NT,
        preferred_element_type=jnp.float32,
    )

    @pl.when(ke < NSTEPS - 1)
    def _acc():
        out_ref[...] += contrib

    @pl.when(ke == NSTEPS - 1)
    def _final():
        out_ref[...] = (out_ref[...] + contrib) * scale_ref[...]


@jax.jit
def kernel(x, W_gate, W_up, W_down):
    B, S, Dm = x.shape
    T = B * S
    xb = x.reshape(T, Dm).astype(jnp.bfloat16)
    wg = W_gate.astype(jnp.bfloat16)

    out = pl.pallas_call(
        _moe_kernel,
        grid=(NSTEPS,),
        in_specs=[
            pl.BlockSpec((T, Dm), lambda ke: (0, 0)),
            pl.BlockSpec((NE, Dm), lambda ke: (0, 0)),
            pl.BlockSpec((KE, Dm), lambda ke: (ke, 0)),
            pl.BlockSpec((Dm, KE), lambda ke: (0, ke)),
        ],
        out_specs=pl.BlockSpec((T, Dm), lambda ke: (0, 0)),
        out_shape=jax.ShapeDtypeStruct((T, Dm), jnp.float32),
        scratch_shapes=[pltpu.VMEM((T, 1), jnp.float32)],
        compiler_params=pltpu.CompilerParams(
            dimension_semantics=("arbitrary",),
        ),
    )(xb, wg, W_up, W_down)
    return out.reshape(B, S, Dm)


# speedup vs baseline: 1.3877x; 1.3877x over previous
import jax
import jax.numpy as jnp
from jax.experimental import pallas as pl
from jax.experimental.pallas import tpu as pltpu

D = 2048
NE = 8
ED = 8192
KE = 2048
NS1 = ED // KE
TM = 256

_NT = (((1,), (1,)), ((), ()))


def _up_kernel(x_ref, wg_ref, wup_ref, h_ref, scale_ref):
    @pl.when(pl.program_id(0) == 0)
    def _gate():
        logits = jax.lax.dot_general(
            x_ref[...], wg_ref[...], _NT, preferred_element_type=jnp.float32
        )
        m = jnp.max(logits, axis=1, keepdims=True)
        e = jnp.exp(logits - m)
        se = jnp.sum(e, axis=1, keepdims=True)
        m1 = jnp.max(e, axis=1, keepdims=True)
        a1 = jnp.argmax(e, axis=1, keepdims=True)
        lane = jax.lax.broadcasted_iota(jnp.int32, e.shape, 1)
        m2 = jnp.max(jnp.where(lane == a1, -jnp.inf, e), axis=1, keepdims=True)
        s = (m1 + m2) / se
        scale_ref[...] = jnp.broadcast_to(s / (s + 1e-9), scale_ref.shape)

    h = jax.lax.dot_general(
        x_ref[...], wup_ref[...], _NT, preferred_element_type=jnp.float32
    )
    h_ref[...] = (h * jax.lax.logistic(h)).astype(jnp.bfloat16)


def _down_kernel(h_ref, wdn_ref, scale_ref, out_ref):
    y = jax.lax.dot_general(
        h_ref[...], wdn_ref[...], _NT, preferred_element_type=jnp.float32
    )
    out_ref[...] = y * scale_ref[:, 0:1]


@jax.jit
def kernel(x, W_gate, W_up, W_down):
    B, S, Dm = x.shape
    T = B * S
    xb = x.reshape(T, Dm).astype(jnp.bfloat16)
    wg = W_gate.astype(jnp.bfloat16)
    wup = W_up.astype(jnp.bfloat16)
    wdn = W_down.astype(jnp.bfloat16)

    h, scale = pl.pallas_call(
        _up_kernel,
        grid=(NS1,),
        in_specs=[
            pl.BlockSpec((T, Dm), lambda i: (0, 0)),
            pl.BlockSpec((NE, Dm), lambda i: (0, 0)),
            pl.BlockSpec((KE, Dm), lambda i: (i, 0)),
        ],
        out_specs=[
            pl.BlockSpec((T, KE), lambda i: (0, i)),
            pl.BlockSpec((T, 128), lambda i: (0, 0)),
        ],
        out_shape=[
            jax.ShapeDtypeStruct((T, ED), jnp.bfloat16),
            jax.ShapeDtypeStruct((T, 128), jnp.float32),
        ],
        compiler_params=pltpu.CompilerParams(
            dimension_semantics=("arbitrary",),
        ),
    )(xb, wg, wup)

    out = pl.pallas_call(
        _down_kernel,
        grid=(T // TM,),
        in_specs=[
            pl.BlockSpec((TM, ED), lambda i: (i, 0)),
            pl.BlockSpec((Dm, ED), lambda i: (0, 0)),
            pl.BlockSpec((TM, 128), lambda i: (i, 0)),
        ],
        out_specs=pl.BlockSpec((TM, Dm), lambda i: (i, 0)),
        out_shape=jax.ShapeDtypeStruct((T, Dm), jnp.float32),
        compiler_params=pltpu.CompilerParams(
            dimension_semantics=("arbitrary",),
        ),
    )(h, wdn, scale)
    return out.reshape(B, S, Dm)


# speedup vs baseline: 1.7695x; 1.2751x over previous
import jax
import jax.numpy as jnp
from jax.experimental import pallas as pl
from jax.experimental.pallas import tpu as pltpu

D = 2048
NE = 8
ED = 8192
KE = 512
NS1 = ED // KE
TM = 256

_NT = (((1,), (1,)), ((), ()))


def _up_kernel(x_ref, wg_ref, wup_ref, wdn_ref, h_ref, wdnb_ref, scale_ref):
    @pl.when(pl.program_id(0) == 0)
    def _gate():
        logits = jax.lax.dot_general(
            x_ref[...], wg_ref[...], _NT, preferred_element_type=jnp.float32
        )
        m = jnp.max(logits, axis=1, keepdims=True)
        e = jnp.exp(logits - m)
        se = jnp.sum(e, axis=1, keepdims=True)
        m1 = jnp.max(e, axis=1, keepdims=True)
        a1 = jnp.argmax(e, axis=1, keepdims=True)
        lane = jax.lax.broadcasted_iota(jnp.int32, e.shape, 1)
        m2 = jnp.max(jnp.where(lane == a1, -jnp.inf, e), axis=1, keepdims=True)
        s = (m1 + m2) / se
        scale_ref[...] = jnp.broadcast_to(s / (s + 1e-9), scale_ref.shape)

    h = jax.lax.dot_general(
        x_ref[...],
        wup_ref[...].astype(jnp.bfloat16),
        _NT,
        preferred_element_type=jnp.float32,
    )
    h_ref[...] = (h * jax.lax.logistic(h)).astype(jnp.bfloat16)
    wdnb_ref[...] = wdn_ref[...].astype(jnp.bfloat16)


def _down_kernel(h_ref, wdn_ref, scale_ref, out_ref):
    y = jax.lax.dot_general(
        h_ref[...], wdn_ref[...], _NT, preferred_element_type=jnp.float32
    )
    out_ref[...] = y * scale_ref[:, 0:1]


@jax.jit
def kernel(x, W_gate, W_up, W_down):
    B, S, Dm = x.shape
    T = B * S
    xb = x.reshape(T, Dm).astype(jnp.bfloat16)
    wg = W_gate.astype(jnp.bfloat16)

    h, wdnb, scale = pl.pallas_call(
        _up_kernel,
        grid=(NS1,),
        in_specs=[
            pl.BlockSpec((T, Dm), lambda i: (0, 0)),
            pl.BlockSpec((NE, Dm), lambda i: (0, 0)),
            pl.BlockSpec((KE, Dm), lambda i: (i, 0)),
            pl.BlockSpec((Dm, KE), lambda i: (0, i)),
        ],
        out_specs=[
            pl.BlockSpec((T, KE), lambda i: (0, i)),
            pl.BlockSpec((Dm, KE), lambda i: (0, i)),
            pl.BlockSpec((T, 128), lambda i: (0, 0)),
        ],
        out_shape=[
            jax.ShapeDtypeStruct((T, ED), jnp.bfloat16),
            jax.ShapeDtypeStruct((Dm, ED), jnp.bfloat16),
            jax.ShapeDtypeStruct((T, 128), jnp.float32),
        ],
        compiler_params=pltpu.CompilerParams(
            dimension_semantics=("arbitrary",),
        ),
    )(xb, wg, W_up, W_down)

    out = pl.pallas_call(
        _down_kernel,
        grid=(T // TM,),
        in_specs=[
            pl.BlockSpec((TM, ED), lambda i: (i, 0)),
            pl.BlockSpec((Dm, ED), lambda i: (0, 0)),
            pl.BlockSpec((TM, 128), lambda i: (i, 0)),
        ],
        out_specs=pl.BlockSpec((TM, Dm), lambda i: (i, 0)),
        out_shape=jax.ShapeDtypeStruct((T, Dm), jnp.float32),
        compiler_params=pltpu.CompilerParams(
            dimension_semantics=("arbitrary",),
        ),
    )(h, wdnb, scale)
    return out.reshape(B, S, Dm)


# speedup vs baseline: 1.8331x; 1.0360x over previous
import jax
import jax.numpy as jnp
from jax.experimental import pallas as pl
from jax.experimental.pallas import tpu as pltpu

D = 2048
NE = 8
ED = 8192
KE = 512
NS1 = ED // KE
TM = 256

_NT = (((1,), (1,)), ((), ()))


def _up_kernel(x_ref, wg_ref, wup_ref, wdn_ref, h_ref, wdnb_ref, scale_ref, xb_ref):
    @pl.when(pl.program_id(0) == 0)
    def _gate():
        xb = x_ref[...].astype(jnp.bfloat16)
        xb_ref[...] = xb
        logits = jax.lax.dot_general(
            xb, wg_ref[...], _NT, preferred_element_type=jnp.float32
        )
        m = jnp.max(logits, axis=1, keepdims=True)
        e = jnp.exp(logits - m)
        se = jnp.sum(e, axis=1, keepdims=True)
        m1 = jnp.max(e, axis=1, keepdims=True)
        a1 = jnp.argmax(e, axis=1, keepdims=True)
        lane = jax.lax.broadcasted_iota(jnp.int32, e.shape, 1)
        m2 = jnp.max(jnp.where(lane == a1, -jnp.inf, e), axis=1, keepdims=True)
        s = (m1 + m2) / se
        scale_ref[...] = jnp.broadcast_to(s / (s + 1e-9), scale_ref.shape)

    h = jax.lax.dot_general(
        xb_ref[...],
        wup_ref[...].astype(jnp.bfloat16),
        _NT,
        preferred_element_type=jnp.float32,
    )
    h_ref[...] = (h * jax.lax.logistic(h)).astype(jnp.bfloat16)
    wdnb_ref[...] = wdn_ref[...].astype(jnp.bfloat16)


def _down_kernel(h_ref, wdn_ref, scale_ref, out_ref):
    y = jax.lax.dot_general(
        h_ref[...], wdn_ref[...], _NT, preferred_element_type=jnp.float32
    )
    out_ref[...] = y * scale_ref[:, 0:1]


@jax.jit
def kernel(x, W_gate, W_up, W_down):
    B, S, Dm = x.shape
    T = B * S
    xf = x.reshape(T, Dm)
    wg = W_gate.astype(jnp.bfloat16)

    h, wdnb, scale = pl.pallas_call(
        _up_kernel,
        grid=(NS1,),
        in_specs=[
            pl.BlockSpec((T, Dm), lambda i: (0, 0)),
            pl.BlockSpec((NE, Dm), lambda i: (0, 0)),
            pl.BlockSpec((KE, Dm), lambda i: (i, 0)),
            pl.BlockSpec((Dm, KE), lambda i: (0, i)),
        ],
        out_specs=[
            pl.BlockSpec((T, KE), lambda i: (0, i)),
            pl.BlockSpec((Dm, KE), lambda i: (0, i)),
            pl.BlockSpec((T, 128), lambda i: (0, 0)),
        ],
        out_shape=[
            jax.ShapeDtypeStruct((T, ED), jnp.bfloat16),
            jax.ShapeDtypeStruct((Dm, ED), jnp.bfloat16),
            jax.ShapeDtypeStruct((T, 128), jnp.float32),
        ],
        scratch_shapes=[pltpu.VMEM((T, Dm), jnp.bfloat16)],
        compiler_params=pltpu.CompilerParams(
            dimension_semantics=("arbitrary",),
        ),
    )(xf, wg, W_up, W_down)

    out = pl.pallas_call(
        _down_kernel,
        grid=(T // TM,),
        in_specs=[
            pl.BlockSpec((TM, ED), lambda i: (i, 0)),
            pl.BlockSpec((Dm, ED), lambda i: (0, 0)),
            pl.BlockSpec((TM, 128), lambda i: (i, 0)),
        ],
        out_specs=pl.BlockSpec((TM, Dm), lambda i: (i, 0)),
        out_shape=jax.ShapeDtypeStruct((T, Dm), jnp.float32),
        compiler_params=pltpu.CompilerParams(
            dimension_semantics=("arbitrary",),
        ),
    )(h, wdnb, scale)
    return out.reshape(B, S, Dm)
